# baseline (speedup 1.0000x reference)
import functools

import jax
import jax.numpy as jnp
from jax import lax
from jax.experimental import pallas as pl
from jax.experimental.pallas import tpu as pltpu
from jax.experimental.pallas import tpu_sc as plsc

_N = 16384
_D = 64
_MARGIN = 0.1
_NE = 9
_NR = 8
_CHUNK = 128
_EGRP = 3
_RGRP = 8
_NB = 8
_BLK = _N // _NB


_RTAB = 1 << 19


def _sc_gather(go_embed, go_rad, eidx, ridx):
    info = plsc.get_sparse_core_info()
    nc, ns = info.num_cores, info.num_subcores
    nw = nc * ns
    e_rows = _NE * _N
    r_rows = _NR * _N
    e_per_w = e_rows // nw
    r_per_w = r_rows // nw
    egroups = e_per_w // (_EGRP * _CHUNK)
    rgroups = r_per_w // (_RGRP * _CHUNK)
    eg_rows = _EGRP * _CHUNK
    rg_rows = _RGRP * _CHUNK

    mesh = plsc.VectorSubcoreMesh(core_axis_name="c", subcore_axis_name="s")

    @functools.partial(
        pl.kernel,
        mesh=mesh,
        compiler_params=pltpu.CompilerParams(use_tc_tiling_on_sc=False),
        out_type=[
            jax.ShapeDtypeStruct((e_rows, _D), jnp.float32),
            jax.ShapeDtypeStruct((r_rows,), jnp.float32),
        ],
        scratch_types=[
            pltpu.VMEM((e_per_w,), jnp.int32),
            pltpu.VMEM((r_per_w,), jnp.int32),
            pltpu.VMEM((eg_rows, _D), jnp.float32),
            pltpu.VMEM((eg_rows, _D), jnp.float32),
            pltpu.VMEM((rg_rows,), jnp.float32),
            pltpu.VMEM((rg_rows,), jnp.float32),
            pltpu.VMEM((_RTAB // ns,), jnp.float32),
            pltpu.VMEM_SHARED((_RTAB,), jnp.float32),
            pltpu.SemaphoreType.DMA,
            pltpu.SemaphoreType.DMA,
        ],
    )
    def k(embed_hbm, rad_hbm, eidx_hbm, ridx_hbm, out_e, out_r,
          eidx_v, ridx_v, ebufa, ebufb, rbufa, rbufb, stage_v, rad_sp,
          sema, semb):
        wid = lax.axis_index("s") * nc + lax.axis_index("c")
        sid = lax.axis_index("s")
        ebase = wid * e_per_w
        rbase = wid * r_per_w
        stg = _RTAB // ns
        soff = sid * stg
        pltpu.sync_copy(rad_hbm.at[pl.ds(soff, stg)], stage_v)
        pltpu.sync_copy(stage_v, rad_sp.at[pl.ds(soff, stg)])
        pltpu.sync_copy(eidx_hbm.at[wid], eidx_v)
        pltpu.sync_copy(ridx_hbm.at[wid], ridx_v)
        plsc.subcore_barrier()

        def e_start(g, buf, sem):
            for b in range(_EGRP):
                c = g * _EGRP + b
                pltpu.make_async_copy(
                    embed_hbm.at[eidx_v.at[pl.ds(c * _CHUNK, _CHUNK)]],
                    buf.at[pl.ds(b * _CHUNK, _CHUNK)], sem).start()

        def e_finish(g, buf, sem):
            for b in range(_EGRP):
                pltpu.make_async_copy(
                    embed_hbm.at[eidx_v.at[pl.ds(0, _CHUNK)]],
                    buf.at[pl.ds(b * _CHUNK, _CHUNK)], sem).wait()
            pltpu.sync_copy(buf, out_e.at[pl.ds(ebase + g * eg_rows, eg_rows)])

        e_start(0, ebufa, sema)

        def ebody(i, carry):
            ga = 2 * i
            gb = 2 * i + 1
            e_start(gb, ebufb, semb)
            e_finish(ga, ebufa, sema)

            @pl.when(gb + 1 < egroups)
            def _():
                e_start(gb + 1, ebufa, sema)

            e_finish(gb, ebufb, semb)
            return carry

        lax.fori_loop(0, egroups // 2, ebody, 0)

        def r_start(g, buf, sem):
            for b in range(_RGRP):
                c = g * _RGRP + b
                pltpu.make_async_copy(
                    rad_sp.at[ridx_v.at[pl.ds(c * _CHUNK, _CHUNK)]],
                    buf.at[pl.ds(b * _CHUNK, _CHUNK)], sem).start()

        def r_finish(g, buf, sem):
            for b in range(_RGRP):
                pltpu.make_async_copy(
                    rad_sp.at[ridx_v.at[pl.ds(0, _CHUNK)]],
                    buf.at[pl.ds(b * _CHUNK, _CHUNK)], sem).wait()
            pltpu.sync_copy(buf, out_r.at[pl.ds(rbase + g * rg_rows, rg_rows)])

        r_start(0, rbufa, sema)

        def rbody(i, carry):
            ga = 2 * i
            gb = 2 * i + 1
            r_start(gb, rbufb, semb)
            r_finish(ga, rbufa, sema)

            @pl.when(gb + 1 < rgroups)
            def _():
                r_start(gb + 1, rbufa, sema)

            r_finish(gb, rbufb, semb)
            return carry

        lax.fori_loop(0, rgroups // 2, rbody, 0)

    return k(go_embed, go_rad, eidx, ridx)


def _tc_body(g_ref, r_ref, rp_ref, rel_ref, i2_ref, i3_ref, gam_ref, bet_ref,
             out_ref, sum_ref, sq_ref, a_ref, b_ref, acc_ref):
    p = pl.program_id(0)
    b = pl.program_id(1)

    @pl.when(jnp.logical_and(p == 0, b == 0))
    def _():
        sum_ref[...] = jnp.zeros_like(sum_ref)
        sq_ref[...] = jnp.zeros_like(sq_ref)

    @pl.when(p == 0)
    def _():
        for j in range(_NE):
            g = g_ref[j]
            sum_ref[j:j + 1, :] = sum_ref[j:j + 1, :] + jnp.sum(
                g, axis=0, keepdims=True)
            sq_ref[j:j + 1, :] = sq_ref[j:j + 1, :] + jnp.sum(
                g * g, axis=0, keepdims=True)

    @pl.when(jnp.logical_and(p == 1, b == 0))
    def _():
        inv_n = jnp.float32(1.0 / _N)
        mu = sum_ref[...] * inv_n
        var = sq_ref[...] * inv_n - mu * mu
        inv = lax.rsqrt(var + 1e-5)
        gam = gam_ref[...]
        bet = bet_ref[...]
        a_ref[...] = inv * gam
        b_ref[...] = bet - mu * inv * gam
        acc_ref[0, 0] = 0.0

    @pl.when(p == 1)
    def _():
        xs = [g_ref[j] * a_ref[j:j + 1, :] + b_ref[j:j + 1, :]
              for j in range(_NE)]
        wi = lax.bitcast_convert_type(r_ref[...], jnp.uint32)
        sel = jnp.where(rp_ref[...] == 1,
                        jnp.bitwise_and(wi, jnp.uint32(0xFFFF0000)),
                        lax.shift_left(wi, jnp.uint32(16)))
        r = jnp.abs(lax.bitcast_convert_type(sel, jnp.float32))

        def dist(u):
            return jnp.sqrt(jnp.sum(u * u, axis=1))

        def relu(t):
            return jnp.maximum(t, 0.0)

        m = jnp.float32(_MARGIN)
        g0 = relu(dist(xs[0] - xs[1]) + r[0] - r[1] - m)
        g1 = (relu(dist(xs[2] - xs[3]) - r[2] - r[3] - m)
              + relu(dist(xs[4] - xs[2]) - r[2] - m)
              + relu(dist(xs[4] - xs[3]) - r[3] - m))
        lanes16 = lax.broadcasted_iota(jnp.int32, (1, 16), 1)
        oh2 = (i2_ref[...] == lanes16).astype(jnp.float32)
        re2 = jnp.dot(oh2, rel_ref[...], preferred_element_type=jnp.float32)
        dst = dist(xs[5] + re2 - xs[6])
        g2 = relu(dst + r[4] - r[5] - m)
        g2n = relu(r[4] + r[5] - dst + m)
        oh3 = (i3_ref[...] == lanes16).astype(jnp.float32)
        re3 = jnp.dot(oh3, rel_ref[...], preferred_element_type=jnp.float32)
        g3 = relu(dist(xs[7] - re3 - xs[8]) - r[6] - r[7] - m)
        acc_ref[0, 0] += jnp.sum(g0 + g1 + g2 + g2n + g3)

    @pl.when(jnp.logical_and(p == 1, b == _NB - 1))
    def _():
        out_ref[...] = jnp.full((1, 1), acc_ref[0, 0] * (1.0 / _N),
                                jnp.float32)


def _tc_loss(g3, r2, rpar, rel, i2, i3, gamma, beta, interpret=False):
    return pl.pallas_call(
        _tc_body,
        grid=(2, _NB),
        in_specs=[
            pl.BlockSpec((_NE, _BLK, _D), lambda p, b: (0, b, 0)),
            pl.BlockSpec((_NR, _BLK), lambda p, b: (0, b)),
            pl.BlockSpec((_NR, _BLK), lambda p, b: (0, b)),
            pl.BlockSpec((16, _D), lambda p, b: (0, 0)),
            pl.BlockSpec((_BLK, 1), lambda p, b: (b, 0)),
            pl.BlockSpec((_BLK, 1), lambda p, b: (b, 0)),
            pl.BlockSpec((1, _D), lambda p, b: (0, 0)),
            pl.BlockSpec((1, _D), lambda p, b: (0, 0)),
        ],
        out_specs=pl.BlockSpec((1, 1), lambda p, b: (0, 0)),
        out_shape=jax.ShapeDtypeStruct((1, 1), jnp.float32),
        scratch_shapes=[
            pltpu.VMEM((_NE, _D), jnp.float32),
            pltpu.VMEM((_NE, _D), jnp.float32),
            pltpu.VMEM((_NE, _D), jnp.float32),
            pltpu.VMEM((_NE, _D), jnp.float32),
            pltpu.SMEM((1, 1), jnp.float32),
        ],
        interpret=interpret,
    )(g3, r2, rpar, rel, i2, i3, gamma, beta)


def kernel(nf0, nf1, nf2, nf3, go_embed, go_rad, rel_embed, bn_gamma, bn_beta):
    eidx = jnp.concatenate([
        nf0[:, 0], nf0[:, 1],
        nf1[:, 0], nf1[:, 1], nf1[:, 2],
        nf2[:, 0], nf2[:, 2],
        nf3[:, 1], nf3[:, 2],
    ])
    ridx = jnp.concatenate([
        nf0[:, 0], nf0[:, 1],
        nf1[:, 0], nf1[:, 1],
        nf2[:, 1], nf2[:, 2],
        nf3[:, 1], nf3[:, 2],
    ])
    info = plsc.get_sparse_core_info()
    nw = info.num_cores * info.num_subcores
    radbf = jnp.pad(go_rad.reshape(-1).astype(jnp.bfloat16),
                    (0, 2 * _RTAB - go_rad.shape[0]))
    radw = lax.bitcast_convert_type(radbf.reshape(_RTAB, 2), jnp.float32)
    ge, gr = _sc_gather(go_embed, radw,
                        eidx.reshape(nw, -1),
                        lax.shift_right_logical(ridx, 1).reshape(nw, -1))
    rpar = jnp.bitwise_and(ridx, 1).reshape(_NR, _N)
    out = _tc_loss(ge.reshape(_NE, _N, _D),
                   gr.reshape(_NR, _N),
                   rpar,
                   rel_embed,
                   nf2[:, 1].reshape(_N, 1),
                   nf3[:, 0].reshape(_N, 1),
                   bn_gamma.reshape(1, _D),
                   bn_beta.reshape(1, _D))
    return out[0, 0]


# speedup vs baseline: 1.1153x; 1.1153x over previous
import functools

import jax
import jax.numpy as jnp
from jax import lax
from jax.experimental import pallas as pl
from jax.experimental.pallas import tpu as pltpu
from jax.experimental.pallas import tpu_sc as plsc

_N = 16384
_D = 64
_MARGIN = 0.1
_NE = 9
_NR = 8
_CHUNK = 128
_EGRP = 3
_RGRP = 8
_NB = 8
_BLK = _N // _NB


_RTAB = 1 << 19


def _sc_gather(go_embed, go_rad, eidx, ridx):
    info = plsc.get_sparse_core_info()
    nc, ns = info.num_cores, info.num_subcores
    nw = nc * ns
    e_rows = _NE * _N
    r_rows = _NR * _N
    e_per_w = e_rows // nw
    r_per_w = r_rows // nw
    egroups = e_per_w // (_EGRP * _CHUNK)
    rgroups = r_per_w // (_RGRP * _CHUNK)
    eg_rows = _EGRP * _CHUNK
    rg_rows = _RGRP * _CHUNK

    mesh = plsc.VectorSubcoreMesh(core_axis_name="c", subcore_axis_name="s")

    @functools.partial(
        pl.kernel,
        mesh=mesh,
        compiler_params=pltpu.CompilerParams(use_tc_tiling_on_sc=False),
        out_type=[
            jax.ShapeDtypeStruct((e_rows, _D), jnp.float32),
            jax.ShapeDtypeStruct((r_rows,), jnp.float32),
        ],
        scratch_types=[
            pltpu.VMEM((e_per_w,), jnp.int32),
            pltpu.VMEM((r_per_w,), jnp.int32),
            pltpu.VMEM((eg_rows, _D), jnp.float32),
            pltpu.VMEM((eg_rows, _D), jnp.float32),
            pltpu.VMEM((rg_rows,), jnp.float32),
            pltpu.VMEM((rg_rows,), jnp.float32),
            pltpu.VMEM((_RTAB // ns,), jnp.float32),
            pltpu.VMEM_SHARED((_RTAB,), jnp.float32),
            pltpu.SemaphoreType.DMA,
            pltpu.SemaphoreType.DMA,
        ],
    )
    def k(embed_hbm, rad_hbm, eidx_hbm, ridx_hbm, out_e, out_r,
          eidx_v, ridx_v, ebufa, ebufb, rbufa, rbufb, stage_v, rad_sp,
          sema, semb):
        wid = lax.axis_index("s") * nc + lax.axis_index("c")
        sid = lax.axis_index("s")
        ebase = wid * e_per_w
        rbase = wid * r_per_w
        stg = _RTAB // ns
        soff = sid * stg
        pltpu.sync_copy(rad_hbm.at[pl.ds(soff, stg)], stage_v)
        pltpu.sync_copy(stage_v, rad_sp.at[pl.ds(soff, stg)])
        pltpu.sync_copy(eidx_hbm.at[wid], eidx_v)
        pltpu.sync_copy(ridx_hbm.at[wid], ridx_v)
        plsc.subcore_barrier()

        def e_start(g, buf, sem):
            for k in range(_CHUNK // 16):
                iv = eidx_v[pl.ds(g * _CHUNK + k * 16, 16)]
                pltpu.make_async_copy(
                    embed_hbm.at[iv],
                    buf.at[pl.ds(k * 16, 16)], sem).start()

        def e_finish(g, buf, sem):
            for k in range(_CHUNK // 16):
                pltpu.make_async_copy(
                    embed_hbm.at[eidx_v[pl.ds(0, 16)]],
                    buf.at[pl.ds(k * 16, 16)], sem).wait()
            pltpu.sync_copy(buf, out_e.at[pl.ds(ebase + g * eg_rows, eg_rows)])

        e_start(0, ebufa, sema)

        def ebody(i, carry):
            ga = 2 * i
            gb = 2 * i + 1
            e_start(gb, ebufb, semb)
            e_finish(ga, ebufa, sema)

            @pl.when(gb + 1 < egroups)
            def _():
                e_start(gb + 1, ebufa, sema)

            e_finish(gb, ebufb, semb)
            return carry

        lax.fori_loop(0, egroups // 2, ebody, 0)

        def r_start(g, buf, sem):
            for k in range(_CHUNK // 16):
                iv = ridx_v[pl.ds(g * _CHUNK + k * 16, 16)]
                pltpu.make_async_copy(
                    rad_sp.at[iv],
                    buf.at[pl.ds(k * 16, 16)], sem).start()

        def r_finish(g, buf, sem):
            for k in range(_CHUNK // 16):
                pltpu.make_async_copy(
                    rad_sp.at[ridx_v[pl.ds(0, 16)]],
                    buf.at[pl.ds(k * 16, 16)], sem).wait()
            pltpu.sync_copy(buf, out_r.at[pl.ds(rbase + g * rg_rows, rg_rows)])

        r_start(0, rbufa, sema)

        def rbody(i, carry):
            ga = 2 * i
            gb = 2 * i + 1
            r_start(gb, rbufb, semb)
            r_finish(ga, rbufa, sema)

            @pl.when(gb + 1 < rgroups)
            def _():
                r_start(gb + 1, rbufa, sema)

            r_finish(gb, rbufb, semb)
            return carry

        lax.fori_loop(0, rgroups // 2, rbody, 0)

    return k(go_embed, go_rad, eidx, ridx)


def _tc_body(g_ref, r_ref, rp_ref, rel_ref, i2_ref, i3_ref, gam_ref, bet_ref,
             out_ref, sum_ref, sq_ref, a_ref, b_ref, acc_ref):
    p = pl.program_id(0)
    b = pl.program_id(1)

    @pl.when(jnp.logical_and(p == 0, b == 0))
    def _():
        sum_ref[...] = jnp.zeros_like(sum_ref)
        sq_ref[...] = jnp.zeros_like(sq_ref)

    @pl.when(p == 0)
    def _():
        for j in range(_NE):
            g = g_ref[j]
            sum_ref[j:j + 1, :] = sum_ref[j:j + 1, :] + jnp.sum(
                g, axis=0, keepdims=True)
            sq_ref[j:j + 1, :] = sq_ref[j:j + 1, :] + jnp.sum(
                g * g, axis=0, keepdims=True)

    @pl.when(jnp.logical_and(p == 1, b == 0))
    def _():
        inv_n = jnp.float32(1.0 / _N)
        mu = sum_ref[...] * inv_n
        var = sq_ref[...] * inv_n - mu * mu
        inv = lax.rsqrt(var + 1e-5)
        gam = gam_ref[...]
        bet = bet_ref[...]
        a_ref[...] = inv * gam
        b_ref[...] = bet - mu * inv * gam
        acc_ref[0, 0] = 0.0

    @pl.when(p == 1)
    def _():
        xs = [g_ref[j] * a_ref[j:j + 1, :] + b_ref[j:j + 1, :]
              for j in range(_NE)]
        wi = lax.bitcast_convert_type(r_ref[...], jnp.uint32)
        sel = jnp.where(rp_ref[...] == 1,
                        jnp.bitwise_and(wi, jnp.uint32(0xFFFF0000)),
                        lax.shift_left(wi, jnp.uint32(16)))
        r = jnp.abs(lax.bitcast_convert_type(sel, jnp.float32))

        def dist(u):
            return jnp.sqrt(jnp.sum(u * u, axis=1))

        def relu(t):
            return jnp.maximum(t, 0.0)

        m = jnp.float32(_MARGIN)
        g0 = relu(dist(xs[0] - xs[1]) + r[0] - r[1] - m)
        g1 = (relu(dist(xs[2] - xs[3]) - r[2] - r[3] - m)
              + relu(dist(xs[4] - xs[2]) - r[2] - m)
              + relu(dist(xs[4] - xs[3]) - r[3] - m))
        lanes16 = lax.broadcasted_iota(jnp.int32, (1, 16), 1)
        oh2 = (i2_ref[...] == lanes16).astype(jnp.float32)
        re2 = jnp.dot(oh2, rel_ref[...], preferred_element_type=jnp.float32)
        dst = dist(xs[5] + re2 - xs[6])
        g2 = relu(dst + r[4] - r[5] - m)
        g2n = relu(r[4] + r[5] - dst + m)
        oh3 = (i3_ref[...] == lanes16).astype(jnp.float32)
        re3 = jnp.dot(oh3, rel_ref[...], preferred_element_type=jnp.float32)
        g3 = relu(dist(xs[7] - re3 - xs[8]) - r[6] - r[7] - m)
        acc_ref[0, 0] += jnp.sum(g0 + g1 + g2 + g2n + g3)

    @pl.when(jnp.logical_and(p == 1, b == _NB - 1))
    def _():
        out_ref[...] = jnp.full((1, 1), acc_ref[0, 0] * (1.0 / _N),
                                jnp.float32)


def _tc_loss(g3, r2, rpar, rel, i2, i3, gamma, beta, interpret=False):
    return pl.pallas_call(
        _tc_body,
        grid=(2, _NB),
        in_specs=[
            pl.BlockSpec((_NE, _BLK, _D), lambda p, b: (0, b, 0)),
            pl.BlockSpec((_NR, _BLK), lambda p, b: (0, b)),
            pl.BlockSpec((_NR, _BLK), lambda p, b: (0, b)),
            pl.BlockSpec((16, _D), lambda p, b: (0, 0)),
            pl.BlockSpec((_BLK, 1), lambda p, b: (b, 0)),
            pl.BlockSpec((_BLK, 1), lambda p, b: (b, 0)),
            pl.BlockSpec((1, _D), lambda p, b: (0, 0)),
            pl.BlockSpec((1, _D), lambda p, b: (0, 0)),
        ],
        out_specs=pl.BlockSpec((1, 1), lambda p, b: (0, 0)),
        out_shape=jax.ShapeDtypeStruct((1, 1), jnp.float32),
        scratch_shapes=[
            pltpu.VMEM((_NE, _D), jnp.float32),
            pltpu.VMEM((_NE, _D), jnp.float32),
            pltpu.VMEM((_NE, _D), jnp.float32),
            pltpu.VMEM((_NE, _D), jnp.float32),
            pltpu.SMEM((1, 1), jnp.float32),
        ],
        interpret=interpret,
    )(g3, r2, rpar, rel, i2, i3, gamma, beta)


def kernel(nf0, nf1, nf2, nf3, go_embed, go_rad, rel_embed, bn_gamma, bn_beta):
    eidx = jnp.concatenate([
        nf0[:, 0], nf0[:, 1],
        nf1[:, 0], nf1[:, 1], nf1[:, 2],
        nf2[:, 0], nf2[:, 2],
        nf3[:, 1], nf3[:, 2],
    ])
    ridx = jnp.concatenate([
        nf0[:, 0], nf0[:, 1],
        nf1[:, 0], nf1[:, 1],
        nf2[:, 1], nf2[:, 2],
        nf3[:, 1], nf3[:, 2],
    ])
    info = plsc.get_sparse_core_info()
    nw = info.num_cores * info.num_subcores
    radbf = jnp.pad(go_rad.reshape(-1).astype(jnp.bfloat16),
                    (0, 2 * _RTAB - go_rad.shape[0]))
    radw = lax.bitcast_convert_type(radbf.reshape(_RTAB, 2), jnp.float32)
    ge, gr = _sc_gather(go_embed, radw,
                        eidx.reshape(nw, -1),
                        lax.shift_right_logical(ridx, 1).reshape(nw, -1))
    rpar = jnp.bitwise_and(ridx, 1).reshape(_NR, _N)
    out = _tc_loss(ge.reshape(_NE, _N, _D),
                   gr.reshape(_NR, _N),
                   rpar,
                   rel_embed,
                   nf2[:, 1].reshape(_N, 1),
                   nf3[:, 0].reshape(_N, 1),
                   bn_gamma.reshape(1, _D),
                   bn_beta.reshape(1, _D))
    return out[0, 0]


# speedup vs baseline: 1.3839x; 1.2408x over previous
import functools

import jax
import jax.numpy as jnp
from jax import lax
from jax.experimental import pallas as pl
from jax.experimental.pallas import tpu as pltpu
from jax.experimental.pallas import tpu_sc as plsc

_N = 16384
_D = 64
_MARGIN = 0.1
_NE = 9
_NR = 8
_CHUNK = 128
_EGRP = 3
_RGRP = 8
_NB = 8
_BLK = _N // _NB


_RTAB = 1 << 19


def _sc_gather(go_embed, go_rad, eidx, ridx):
    info = plsc.get_sparse_core_info()
    nc, ns = info.num_cores, info.num_subcores
    nw = nc * ns
    e_rows = _NE * _N
    r_rows = _NR * _N
    e_per_w = e_rows // nw
    r_per_w = r_rows // nw
    egroups = e_per_w // (_EGRP * _CHUNK)
    rgroups = r_per_w // (_RGRP * _CHUNK)
    eg_rows = _EGRP * _CHUNK
    rg_rows = _RGRP * _CHUNK

    mesh = plsc.VectorSubcoreMesh(core_axis_name="c", subcore_axis_name="s")

    @functools.partial(
        pl.kernel,
        mesh=mesh,
        compiler_params=pltpu.CompilerParams(use_tc_tiling_on_sc=False),
        out_type=[
            jax.ShapeDtypeStruct((e_rows, _D), jnp.float32),
            jax.ShapeDtypeStruct((r_rows,), jnp.float32),
        ],
        scratch_types=[
            pltpu.VMEM((e_per_w,), jnp.int32),
            pltpu.VMEM((r_per_w,), jnp.int32),
            pltpu.VMEM((eg_rows, _D), jnp.float32),
            pltpu.VMEM((eg_rows, _D), jnp.float32),
            pltpu.VMEM((rg_rows,), jnp.float32),
            pltpu.VMEM((rg_rows,), jnp.float32),
            pltpu.SemaphoreType.DMA,
            pltpu.SemaphoreType.DMA,
        ],
    )
    def k(embed_hbm, rad_hbm, eidx_hbm, ridx_hbm, out_e, out_r,
          eidx_v, ridx_v, ebufa, ebufb, rbufa, rbufb, sema, semb):
        wid = lax.axis_index("s") * nc + lax.axis_index("c")
        ebase = wid * e_per_w
        rbase = wid * r_per_w
        pltpu.sync_copy(eidx_hbm.at[wid], eidx_v)
        pltpu.sync_copy(ridx_hbm.at[wid], ridx_v)

        def e_start(g, buf, sem):
            for k in range(_CHUNK // 16):
                iv = eidx_v[pl.ds(g * _CHUNK + k * 16, 16)]
                pltpu.make_async_copy(
                    embed_hbm.at[iv],
                    buf.at[pl.ds(k * 16, 16)], sem).start()

        def e_finish(g, buf, sem):
            for k in range(_CHUNK // 16):
                pltpu.make_async_copy(
                    embed_hbm.at[eidx_v[pl.ds(0, 16)]],
                    buf.at[pl.ds(k * 16, 16)], sem).wait()
            pltpu.sync_copy(buf, out_e.at[pl.ds(ebase + g * eg_rows, eg_rows)])

        e_start(0, ebufa, sema)

        def ebody(i, carry):
            ga = 2 * i
            gb = 2 * i + 1
            e_start(gb, ebufb, semb)
            e_finish(ga, ebufa, sema)

            @pl.when(gb + 1 < egroups)
            def _():
                e_start(gb + 1, ebufa, sema)

            e_finish(gb, ebufb, semb)
            return carry

        lax.fori_loop(0, egroups // 2, ebody, 0)

        def r_start(g, buf, sem):
            for k in range(_CHUNK // 16):
                iv = ridx_v[pl.ds(g * _CHUNK + k * 16, 16)]
                pltpu.make_async_copy(
                    rad_hbm.at[iv],
                    buf.at[pl.ds(k * 16, 16)], sem).start()

        def r_finish(g, buf, sem):
            for k in range(_CHUNK // 16):
                pltpu.make_async_copy(
                    rad_hbm.at[ridx_v[pl.ds(0, 16)]],
                    buf.at[pl.ds(k * 16, 16)], sem).wait()
            pltpu.sync_copy(buf, out_r.at[pl.ds(rbase + g * rg_rows, rg_rows)])

        r_start(0, rbufa, sema)

        def rbody(i, carry):
            ga = 2 * i
            gb = 2 * i + 1
            r_start(gb, rbufb, semb)
            r_finish(ga, rbufa, sema)

            @pl.when(gb + 1 < rgroups)
            def _():
                r_start(gb + 1, rbufa, sema)

            r_finish(gb, rbufb, semb)
            return carry

        lax.fori_loop(0, rgroups // 2, rbody, 0)

    return k(go_embed, go_rad, eidx, ridx)


def _tc_body(g_ref, r_ref, rel_ref, i2_ref, i3_ref, gam_ref, bet_ref,
             out_ref, sum_ref, sq_ref, a_ref, b_ref, acc_ref):
    p = pl.program_id(0)
    b = pl.program_id(1)

    @pl.when(jnp.logical_and(p == 0, b == 0))
    def _():
        sum_ref[...] = jnp.zeros_like(sum_ref)
        sq_ref[...] = jnp.zeros_like(sq_ref)

    @pl.when(p == 0)
    def _():
        for j in range(_NE):
            g = g_ref[j]
            sum_ref[j:j + 1, :] = sum_ref[j:j + 1, :] + jnp.sum(
                g, axis=0, keepdims=True)
            sq_ref[j:j + 1, :] = sq_ref[j:j + 1, :] + jnp.sum(
                g * g, axis=0, keepdims=True)

    @pl.when(jnp.logical_and(p == 1, b == 0))
    def _():
        inv_n = jnp.float32(1.0 / _N)
        mu = sum_ref[...] * inv_n
        var = sq_ref[...] * inv_n - mu * mu
        inv = lax.rsqrt(var + 1e-5)
        gam = gam_ref[...]
        bet = bet_ref[...]
        a_ref[...] = inv * gam
        b_ref[...] = bet - mu * inv * gam
        acc_ref[0, 0] = 0.0

    @pl.when(p == 1)
    def _():
        xs = [g_ref[j] * a_ref[j:j + 1, :] + b_ref[j:j + 1, :]
              for j in range(_NE)]
        r = jnp.abs(r_ref[...])

        def dist(u):
            return jnp.sqrt(jnp.sum(u * u, axis=1))

        def relu(t):
            return jnp.maximum(t, 0.0)

        m = jnp.float32(_MARGIN)
        g0 = relu(dist(xs[0] - xs[1]) + r[0] - r[1] - m)
        g1 = (relu(dist(xs[2] - xs[3]) - r[2] - r[3] - m)
              + relu(dist(xs[4] - xs[2]) - r[2] - m)
              + relu(dist(xs[4] - xs[3]) - r[3] - m))
        lanes16 = lax.broadcasted_iota(jnp.int32, (1, 16), 1)
        oh2 = (i2_ref[...] == lanes16).astype(jnp.float32)
        re2 = jnp.dot(oh2, rel_ref[...], preferred_element_type=jnp.float32)
        dst = dist(xs[5] + re2 - xs[6])
        g2 = relu(dst + r[4] - r[5] - m)
        g2n = relu(r[4] + r[5] - dst + m)
        oh3 = (i3_ref[...] == lanes16).astype(jnp.float32)
        re3 = jnp.dot(oh3, rel_ref[...], preferred_element_type=jnp.float32)
        g3 = relu(dist(xs[7] - re3 - xs[8]) - r[6] - r[7] - m)
        acc_ref[0, 0] += jnp.sum(g0 + g1 + g2 + g2n + g3)

    @pl.when(jnp.logical_and(p == 1, b == _NB - 1))
    def _():
        out_ref[...] = jnp.full((1, 1), acc_ref[0, 0] * (1.0 / _N),
                                jnp.float32)


def _tc_loss(g3, r2, rel, i2, i3, gamma, beta, interpret=False):
    return pl.pallas_call(
        _tc_body,
        grid=(2, _NB),
        in_specs=[
            pl.BlockSpec((_NE, _BLK, _D), lambda p, b: (0, b, 0)),
            pl.BlockSpec((_NR, _BLK), lambda p, b: (0, b)),
            pl.BlockSpec((16, _D), lambda p, b: (0, 0)),
            pl.BlockSpec((_BLK, 1), lambda p, b: (b, 0)),
            pl.BlockSpec((_BLK, 1), lambda p, b: (b, 0)),
            pl.BlockSpec((1, _D), lambda p, b: (0, 0)),
            pl.BlockSpec((1, _D), lambda p, b: (0, 0)),
        ],
        out_specs=pl.BlockSpec((1, 1), lambda p, b: (0, 0)),
        out_shape=jax.ShapeDtypeStruct((1, 1), jnp.float32),
        scratch_shapes=[
            pltpu.VMEM((_NE, _D), jnp.float32),
            pltpu.VMEM((_NE, _D), jnp.float32),
            pltpu.VMEM((_NE, _D), jnp.float32),
            pltpu.VMEM((_NE, _D), jnp.float32),
            pltpu.SMEM((1, 1), jnp.float32),
        ],
        interpret=interpret,
    )(g3, r2, rel, i2, i3, gamma, beta)


def kernel(nf0, nf1, nf2, nf3, go_embed, go_rad, rel_embed, bn_gamma, bn_beta):
    eidx = jnp.concatenate([
        nf0[:, 0], nf0[:, 1],
        nf1[:, 0], nf1[:, 1], nf1[:, 2],
        nf2[:, 0], nf2[:, 2],
        nf3[:, 1], nf3[:, 2],
    ])
    ridx = jnp.concatenate([
        nf0[:, 0], nf0[:, 1],
        nf1[:, 0], nf1[:, 1],
        nf2[:, 1], nf2[:, 2],
        nf3[:, 1], nf3[:, 2],
    ])
    info = plsc.get_sparse_core_info()
    nw = info.num_cores * info.num_subcores
    ge, gr = _sc_gather(go_embed, go_rad.reshape(-1),
                        eidx.reshape(nw, -1),
                        ridx.reshape(nw, -1))
    out = _tc_loss(ge.reshape(_NE, _N, _D),
                   gr.reshape(_NR, _N),
                   rel_embed,
                   nf2[:, 1].reshape(_N, 1),
                   nf3[:, 0].reshape(_N, 1),
                   bn_gamma.reshape(1, _D),
                   bn_beta.reshape(1, _D))
    return out[0, 0]
